# baseline (speedup 1.0000x reference)
import functools
import jax
import jax.numpy as jnp
from jax import lax
from jax.experimental import pallas as pl
from jax.experimental.pallas import tpu as pltpu
from jax.experimental.pallas import tpu_sc as plsc

N, D, E, H, TOPK = 8192, 768, 8, 512, 2
TB = 256
B = 256
DW = D // 2
P = N * TOPK + E * B
NB = P // B

NC, NS = 2, 16
NW = NC * NS
TPW = N // NW
CK = 64
NCH = TPW // CK


def _gelu(v):
    return 0.5 * v * (1.0 + jax.lax.erf(v * 0.7071067811865476))


def _top2(logits):
    T = logits.shape[0]
    iota = jax.lax.broadcasted_iota(jnp.int32, (T, E), 1)
    m1 = jnp.max(logits, axis=-1, keepdims=True)
    i1 = jnp.min(jnp.where(logits == m1, iota, E), axis=-1, keepdims=True)
    l2 = jnp.where(iota == i1, -jnp.inf, logits)
    m2 = jnp.max(l2, axis=-1, keepdims=True)
    i2 = jnp.min(jnp.where(l2 == m2, iota, E), axis=-1, keepdims=True)
    r = jnp.exp(m2 - m1)
    w_a = 1.0 / (1.0 + r)
    return w_a, 1.0 - w_a, i1, i2


def _gate_body(x_ref, gw_ref, wts_ref, eid_ref, rnk_ref, cnts_ref, xh_ref, cntv):
    i = pl.program_id(0)

    @pl.when(i == 0)
    def _init():
        cntv[...] = jnp.zeros((1, E), jnp.float32)

    xh_ref[...] = x_ref[...].astype(jnp.bfloat16)
    logits = jnp.dot(x_ref[...], gw_ref[...], preferred_element_type=jnp.float32)
    w_a, w_b, i1, i2 = _top2(logits)
    wts_ref[...] = jnp.concatenate([w_a, w_b], axis=1)
    eid_ref[...] = jnp.concatenate([i1, i2], axis=1)

    laneio = jax.lax.broadcasted_iota(jnp.int32, (TB, E), 1)
    cnt_tok = (laneio == i1).astype(jnp.float32) + (laneio == i2).astype(jnp.float32)
    lt = (jax.lax.broadcasted_iota(jnp.int32, (TB, TB), 0)
          > jax.lax.broadcasted_iota(jnp.int32, (TB, TB), 1)).astype(jnp.float32)
    cumexc = jnp.dot(lt, cnt_tok, preferred_element_type=jnp.float32)
    base = cntv[...]
    r_all = cumexc + base
    rankA = jnp.sum(jnp.where(laneio == i1, r_all, 0.0), axis=1, keepdims=True)
    rankB = jnp.sum(jnp.where(laneio == i2, r_all, 0.0), axis=1, keepdims=True)
    rnk_ref[...] = jnp.concatenate([rankA, rankB], axis=1).astype(jnp.int32)

    newtot = base + jnp.sum(cnt_tok, axis=0, keepdims=True)
    cntv[...] = newtot
    cnts_ref[...] = jnp.broadcast_to(newtot, (E, E)).astype(jnp.int32)


def _gate(x, gate_W):
    return pl.pallas_call(
        _gate_body,
        grid=(N // TB,),
        in_specs=[
            pl.BlockSpec((TB, D), lambda i: (i, 0)),
            pl.BlockSpec((D, E), lambda i: (0, 0)),
        ],
        out_specs=[
            pl.BlockSpec((TB, TOPK), lambda i: (i, 0)),
            pl.BlockSpec((TB, TOPK), lambda i: (i, 0)),
            pl.BlockSpec((TB, TOPK), lambda i: (i, 0)),
            pl.BlockSpec((E, E), lambda i: (0, 0)),
            pl.BlockSpec((TB, D), lambda i: (i, 0)),
        ],
        out_shape=[
            jax.ShapeDtypeStruct((N, TOPK), jnp.float32),
            jax.ShapeDtypeStruct((N, TOPK), jnp.int32),
            jax.ShapeDtypeStruct((N, TOPK), jnp.int32),
            jax.ShapeDtypeStruct((E, E), jnp.int32),
            jax.ShapeDtypeStruct((N, D), jnp.bfloat16),
        ],
        scratch_shapes=[pltpu.VMEM((1, E), jnp.float32)],
    )(x, gate_W)


def _route_body(eid_ref, rnk_ref, cnts_ref, pa_ref, pb_ref, bexp_ref):
    c = cnts_ref[0:1, :]
    padded = jnp.bitwise_and(c + (B - 1), ~(B - 1))
    ut = (jax.lax.broadcasted_iota(jnp.int32, (E, E), 0)
          < jax.lax.broadcasted_iota(jnp.int32, (E, E), 1)).astype(jnp.float32)
    off = jnp.dot(padded.astype(jnp.float32), ut,
                  preferred_element_type=jnp.float32).astype(jnp.int32)
    eid = eid_ref[...]
    laneio = jax.lax.broadcasted_iota(jnp.int32, (TB, E), 1)
    offb = jnp.broadcast_to(off, (TB, E))
    offA = jnp.sum(jnp.where(laneio == eid[:, 0:1], offb, 0),
                   axis=1, keepdims=True)
    offB = jnp.sum(jnp.where(laneio == eid[:, 1:2], offb, 0),
                   axis=1, keepdims=True)
    rnk = rnk_ref[...]
    pa_ref[...] = offA + rnk[:, 0:1]
    pb_ref[...] = offB + rnk[:, 1:2]
    jb = jax.lax.broadcasted_iota(jnp.int32, (NB, E), 0) * B
    used = (jb >= jnp.broadcast_to(off, (NB, E))).astype(jnp.int32)
    bexp_ref[...] = jnp.sum(used, axis=1, keepdims=True) - 1


def _route(eid, rnk, cnts8):
    return pl.pallas_call(
        _route_body,
        grid=(N // TB,),
        in_specs=[
            pl.BlockSpec((TB, TOPK), lambda i: (i, 0)),
            pl.BlockSpec((TB, TOPK), lambda i: (i, 0)),
            pl.BlockSpec((E, E), lambda i: (0, 0)),
        ],
        out_specs=[
            pl.BlockSpec((TB, 1), lambda i: (i, 0)),
            pl.BlockSpec((TB, 1), lambda i: (i, 0)),
            pl.BlockSpec((NB, 1), lambda i: (0, 0)),
        ],
        out_shape=[
            jax.ShapeDtypeStruct((N, 1), jnp.int32),
            jax.ShapeDtypeStruct((N, 1), jnp.int32),
            jax.ShapeDtypeStruct((NB, 1), jnp.int32),
        ],
    )(eid, rnk, cnts8)


def _sc_mesh():
    return plsc.VectorSubcoreMesh(core_axis_name="c", subcore_axis_name="s")


def _dispatch(x, posA, posB):
    @functools.partial(
        pl.kernel,
        mesh=_sc_mesh(),
        out_type=jax.ShapeDtypeStruct((P, DW), jnp.int32),
        scratch_types=[
            pltpu.VMEM((CK, DW), jnp.int32),
            pltpu.VMEM((CK,), jnp.int32),
            pltpu.VMEM((CK,), jnp.int32),
            pltpu.SemaphoreType.DMA,
            pltpu.SemaphoreType.DMA,
        ],
    )
    def body(x_hbm, pa_hbm, pb_hbm, xd_hbm, xv, pva, pvb, semA, semB):
        wid = lax.axis_index("s") * NC + lax.axis_index("c")
        base = wid * TPW
        for i in range(NCH):
            s = base + i * CK
            pltpu.sync_copy(x_hbm.at[pl.ds(s, CK)], xv)
            pltpu.sync_copy(pa_hbm.at[pl.ds(s, CK)], pva)
            pltpu.sync_copy(pb_hbm.at[pl.ds(s, CK)], pvb)
            cpA = pltpu.async_copy(xv, xd_hbm.at[pva], semA)
            cpB = pltpu.async_copy(xv, xd_hbm.at[pvb], semB)
            cpA.wait()
            cpB.wait()

    return body(x, posA, posB)


def _combine_gather(yd, posA, posB):
    @functools.partial(
        pl.kernel,
        mesh=_sc_mesh(),
        out_type=[
            jax.ShapeDtypeStruct((N, DW), jnp.int32),
            jax.ShapeDtypeStruct((N, DW), jnp.int32),
        ],
        scratch_types=[
            pltpu.VMEM((CK, DW), jnp.int32),
            pltpu.VMEM((CK, DW), jnp.int32),
            pltpu.VMEM((CK,), jnp.int32),
            pltpu.VMEM((CK,), jnp.int32),
            pltpu.SemaphoreType.DMA,
            pltpu.SemaphoreType.DMA,
        ],
    )
    def body(yd_hbm, pa_hbm, pb_hbm, ya_hbm, yb_hbm, yva, yvb, pva, pvb,
             semA, semB):
        wid = lax.axis_index("s") * NC + lax.axis_index("c")
        base = wid * TPW
        for i in range(NCH):
            s = base + i * CK
            pltpu.sync_copy(pa_hbm.at[pl.ds(s, CK)], pva)
            pltpu.sync_copy(pb_hbm.at[pl.ds(s, CK)], pvb)
            cpA = pltpu.async_copy(yd_hbm.at[pva], yva, semA)
            cpB = pltpu.async_copy(yd_hbm.at[pvb], yvb, semB)
            cpA.wait()
            cpB.wait()
            pltpu.sync_copy(yva, ya_hbm.at[pl.ds(s, CK)])
            pltpu.sync_copy(yvb, yb_hbm.at[pl.ds(s, CK)])

    return body(yd, posA, posB)


def _ffn_body(bexp_ref, xd_ref, w1_ref, b1_ref, w2_ref, b2_ref, yd_ref):
    xb = xd_ref[...].astype(jnp.float32)
    h = _gelu(jnp.dot(xb, w1_ref[0], preferred_element_type=jnp.float32)
              + b1_ref[0])
    y = (jnp.dot(h, w2_ref[0], preferred_element_type=jnp.float32)
         + b2_ref[0])
    yd_ref[...] = y.astype(jnp.bfloat16)


def _ffn(xd, W1, b1, W2, b2, bexp):
    grid_spec = pltpu.PrefetchScalarGridSpec(
        num_scalar_prefetch=1,
        grid=(NB,),
        in_specs=[
            pl.BlockSpec((B, D), lambda i, be: (i, 0)),
            pl.BlockSpec((1, D, H), lambda i, be: (be[i], 0, 0)),
            pl.BlockSpec((1, 1, H), lambda i, be: (be[i], 0, 0)),
            pl.BlockSpec((1, H, D), lambda i, be: (be[i], 0, 0)),
            pl.BlockSpec((1, 1, D), lambda i, be: (be[i], 0, 0)),
        ],
        out_specs=pl.BlockSpec((B, D), lambda i, be: (i, 0)),
    )
    return pl.pallas_call(
        _ffn_body,
        grid_spec=grid_spec,
        out_shape=jax.ShapeDtypeStruct((P, D), jnp.bfloat16),
    )(bexp, xd, W1, b1.reshape(E, 1, H), W2, b2.reshape(E, 1, D))


def _finish_body(x_ref, ya_ref, yb_ref, wts_ref, ws_ref, bs_ref, g_ref,
                 be_ref, o_ref):
    shared = _gelu(jnp.dot(x_ref[...], ws_ref[...],
                           preferred_element_type=jnp.float32)
                   + bs_ref[...][None, :])
    w = wts_ref[...]
    out = (w[:, 0:1] * ya_ref[...].astype(jnp.float32)
           + w[:, 1:2] * yb_ref[...].astype(jnp.float32)
           + 0.5 * shared)
    mu = jnp.mean(out, axis=-1, keepdims=True)
    d = out - mu
    var = jnp.mean(d * d, axis=-1, keepdims=True)
    o_ref[...] = d * jax.lax.rsqrt(var + 1e-5) * g_ref[...][None, :] + be_ref[...][None, :]


def _finish(x, yA, yB, wts, Ws, bs, gamma, beta):
    full = lambda shape: pl.BlockSpec(shape, lambda i: (0,) * len(shape))
    return pl.pallas_call(
        _finish_body,
        grid=(N // TB,),
        in_specs=[
            pl.BlockSpec((TB, D), lambda i: (i, 0)),
            pl.BlockSpec((TB, D), lambda i: (i, 0)),
            pl.BlockSpec((TB, D), lambda i: (i, 0)),
            pl.BlockSpec((TB, TOPK), lambda i: (i, 0)),
            full((D, D)),
            full((D,)),
            full((D,)),
            full((D,)),
        ],
        out_specs=pl.BlockSpec((TB, D), lambda i: (i, 0)),
        out_shape=jax.ShapeDtypeStruct((N, D), jnp.float32),
    )(x, yA, yB, wts, Ws, bs, gamma, beta)


def _as_i32(a):
    r = a.shape[0]
    return jax.lax.bitcast_convert_type(a.reshape(r, DW, 2), jnp.int32)


def _as_bf16(a):
    r = a.shape[0]
    return jax.lax.bitcast_convert_type(a, jnp.bfloat16).reshape(r, D)


@jax.jit
def kernel(x, gate_W, W1, b1, W2, b2, Ws, bs, gamma, beta):
    wts, eid, rnk, cnts8, xh = _gate(x, gate_W)
    posA2, posB2, bexp2 = _route(eid, rnk, cnts8)
    posA = posA2.reshape(N)
    posB = posB2.reshape(N)
    bexp = bexp2.reshape(NB)
    xd32 = _dispatch(_as_i32(xh), posA, posB)
    yd = _ffn(_as_bf16(xd32), W1, b1, W2, b2, bexp)
    yA32, yB32 = _combine_gather(_as_i32(yd), posA, posB)
    return _finish(x, _as_bf16(yA32), _as_bf16(yB32), wts, Ws, bs, gamma, beta)


# speedup vs baseline: 4.4211x; 4.4211x over previous
import functools
import jax
import jax.numpy as jnp
from jax import lax
from jax.experimental import pallas as pl
from jax.experimental.pallas import tpu as pltpu
from jax.experimental.pallas import tpu_sc as plsc

N, D, E, H, TOPK = 8192, 768, 8, 512, 2
TB = 256
B = 256
DW = D // 2
P = N * TOPK + E * B
NB = P // B

NC, NS = 2, 16
NW = NC * NS
TPW = N // NW
CK = 64
NCH = TPW // CK


def _gelu(v):
    return 0.5 * v * (1.0 + jax.lax.erf(v * 0.7071067811865476))


def _pack_rows(xf):
    u = jax.lax.bitcast_convert_type(xf, jnp.uint32)
    r = u + 0x7FFF + ((u >> 16) & 1)
    r = (r >> 16).astype(jnp.int32)
    lo = r[:, :DW]
    hi = r[:, DW:]
    return jnp.bitwise_or(lo, hi << 16)


def _unpack_rows(w):
    lo = jax.lax.bitcast_convert_type(w << 16, jnp.float32)
    hi = jax.lax.bitcast_convert_type(
        jnp.bitwise_and(w, jnp.int32(-65536)), jnp.float32)
    return jnp.concatenate([lo, hi], axis=1)


def _top2(logits):
    T = logits.shape[0]
    iota = jax.lax.broadcasted_iota(jnp.int32, (T, E), 1)
    m1 = jnp.max(logits, axis=-1, keepdims=True)
    i1 = jnp.min(jnp.where(logits == m1, iota, E), axis=-1, keepdims=True)
    l2 = jnp.where(iota == i1, -jnp.inf, logits)
    m2 = jnp.max(l2, axis=-1, keepdims=True)
    i2 = jnp.min(jnp.where(l2 == m2, iota, E), axis=-1, keepdims=True)
    r = jnp.exp(m2 - m1)
    w_a = 1.0 / (1.0 + r)
    return w_a, 1.0 - w_a, i1, i2


def _gate_body(x_ref, gw_ref, wts_ref, eid_ref, rnk_ref, cnts_ref, xh_ref, cntv):
    i = pl.program_id(0)

    @pl.when(i == 0)
    def _init():
        cntv[...] = jnp.zeros((1, E), jnp.float32)

    xh_ref[...] = _pack_rows(x_ref[...])
    logits = jnp.dot(x_ref[...], gw_ref[...], preferred_element_type=jnp.float32)
    w_a, w_b, i1, i2 = _top2(logits)
    wts_ref[...] = jnp.concatenate([w_a, w_b], axis=1)
    eid_ref[...] = jnp.concatenate([i1, i2], axis=1)

    laneio = jax.lax.broadcasted_iota(jnp.int32, (TB, E), 1)
    cnt_tok = (laneio == i1).astype(jnp.float32) + (laneio == i2).astype(jnp.float32)
    lt = (jax.lax.broadcasted_iota(jnp.int32, (TB, TB), 0)
          > jax.lax.broadcasted_iota(jnp.int32, (TB, TB), 1)).astype(jnp.float32)
    cumexc = jnp.dot(lt, cnt_tok, preferred_element_type=jnp.float32)
    base = cntv[...]
    r_all = cumexc + base
    rankA = jnp.sum(jnp.where(laneio == i1, r_all, 0.0), axis=1, keepdims=True)
    rankB = jnp.sum(jnp.where(laneio == i2, r_all, 0.0), axis=1, keepdims=True)
    rnk_ref[...] = jnp.concatenate([rankA, rankB], axis=1).astype(jnp.int32)

    newtot = base + jnp.sum(cnt_tok, axis=0, keepdims=True)
    cntv[...] = newtot
    cnts_ref[...] = jnp.broadcast_to(newtot, (E, E)).astype(jnp.int32)


def _gate(x, gate_W):
    return pl.pallas_call(
        _gate_body,
        grid=(N // TB,),
        in_specs=[
            pl.BlockSpec((TB, D), lambda i: (i, 0)),
            pl.BlockSpec((D, E), lambda i: (0, 0)),
        ],
        out_specs=[
            pl.BlockSpec((TB, TOPK), lambda i: (i, 0)),
            pl.BlockSpec((TB, TOPK), lambda i: (i, 0)),
            pl.BlockSpec((TB, TOPK), lambda i: (i, 0)),
            pl.BlockSpec((E, E), lambda i: (0, 0)),
            pl.BlockSpec((TB, DW), lambda i: (i, 0)),
        ],
        out_shape=[
            jax.ShapeDtypeStruct((N, TOPK), jnp.float32),
            jax.ShapeDtypeStruct((N, TOPK), jnp.int32),
            jax.ShapeDtypeStruct((N, TOPK), jnp.int32),
            jax.ShapeDtypeStruct((E, E), jnp.int32),
            jax.ShapeDtypeStruct((N, DW), jnp.int32),
        ],
        scratch_shapes=[pltpu.VMEM((1, E), jnp.float32)],
    )(x, gate_W)


def _route_body(eid_ref, rnk_ref, cnts_ref, pa_ref, pb_ref, bexp_ref):
    c = cnts_ref[0:1, :]
    padded = jnp.bitwise_and(c + (B - 1), ~(B - 1))
    ut = (jax.lax.broadcasted_iota(jnp.int32, (E, E), 0)
          < jax.lax.broadcasted_iota(jnp.int32, (E, E), 1)).astype(jnp.float32)
    off = jnp.dot(padded.astype(jnp.float32), ut,
                  preferred_element_type=jnp.float32).astype(jnp.int32)
    eid = eid_ref[...]
    laneio = jax.lax.broadcasted_iota(jnp.int32, (TB, E), 1)
    offb = jnp.broadcast_to(off, (TB, E))
    offA = jnp.sum(jnp.where(laneio == eid[:, 0:1], offb, 0),
                   axis=1, keepdims=True)
    offB = jnp.sum(jnp.where(laneio == eid[:, 1:2], offb, 0),
                   axis=1, keepdims=True)
    rnk = rnk_ref[...]
    pa_ref[...] = offA + rnk[:, 0:1]
    pb_ref[...] = offB + rnk[:, 1:2]
    jb = jax.lax.broadcasted_iota(jnp.int32, (NB, E), 0) * B
    used = (jb >= jnp.broadcast_to(off, (NB, E))).astype(jnp.int32)
    bexp_ref[...] = jnp.sum(used, axis=1, keepdims=True) - 1


def _route(eid, rnk, cnts8):
    return pl.pallas_call(
        _route_body,
        grid=(N // TB,),
        in_specs=[
            pl.BlockSpec((TB, TOPK), lambda i: (i, 0)),
            pl.BlockSpec((TB, TOPK), lambda i: (i, 0)),
            pl.BlockSpec((E, E), lambda i: (0, 0)),
        ],
        out_specs=[
            pl.BlockSpec((TB, 1), lambda i: (i, 0)),
            pl.BlockSpec((TB, 1), lambda i: (i, 0)),
            pl.BlockSpec((NB, 1), lambda i: (0, 0)),
        ],
        out_shape=[
            jax.ShapeDtypeStruct((N, 1), jnp.int32),
            jax.ShapeDtypeStruct((N, 1), jnp.int32),
            jax.ShapeDtypeStruct((NB, 1), jnp.int32),
        ],
    )(eid, rnk, cnts8)


def _sc_mesh():
    return plsc.VectorSubcoreMesh(core_axis_name="c", subcore_axis_name="s")


def _dispatch(x, posA, posB):
    @functools.partial(
        pl.kernel,
        mesh=_sc_mesh(),
        out_type=jax.ShapeDtypeStruct((P, DW), jnp.int32),
        scratch_types=[
            pltpu.VMEM((CK, DW), jnp.int32),
            pltpu.VMEM((CK,), jnp.int32),
            pltpu.VMEM((CK,), jnp.int32),
            pltpu.SemaphoreType.DMA,
            pltpu.SemaphoreType.DMA,
        ],
    )
    def body(x_hbm, pa_hbm, pb_hbm, xd_hbm, xv, pva, pvb, semA, semB):
        wid = lax.axis_index("s") * NC + lax.axis_index("c")
        base = wid * TPW
        for i in range(NCH):
            s = base + i * CK
            pltpu.sync_copy(x_hbm.at[pl.ds(s, CK)], xv)
            pltpu.sync_copy(pa_hbm.at[pl.ds(s, CK)], pva)
            pltpu.sync_copy(pb_hbm.at[pl.ds(s, CK)], pvb)
            cpA = pltpu.async_copy(xv, xd_hbm.at[pva], semA)
            cpB = pltpu.async_copy(xv, xd_hbm.at[pvb], semB)
            cpA.wait()
            cpB.wait()

    return body(x, posA, posB)


def _combine_gather(yd, posA, posB):
    @functools.partial(
        pl.kernel,
        mesh=_sc_mesh(),
        out_type=[
            jax.ShapeDtypeStruct((N, DW), jnp.int32),
            jax.ShapeDtypeStruct((N, DW), jnp.int32),
        ],
        scratch_types=[
            pltpu.VMEM((CK, DW), jnp.int32),
            pltpu.VMEM((CK, DW), jnp.int32),
            pltpu.VMEM((CK,), jnp.int32),
            pltpu.VMEM((CK,), jnp.int32),
            pltpu.SemaphoreType.DMA,
            pltpu.SemaphoreType.DMA,
        ],
    )
    def body(yd_hbm, pa_hbm, pb_hbm, ya_hbm, yb_hbm, yva, yvb, pva, pvb,
             semA, semB):
        wid = lax.axis_index("s") * NC + lax.axis_index("c")
        base = wid * TPW
        for i in range(NCH):
            s = base + i * CK
            pltpu.sync_copy(pa_hbm.at[pl.ds(s, CK)], pva)
            pltpu.sync_copy(pb_hbm.at[pl.ds(s, CK)], pvb)
            cpA = pltpu.async_copy(yd_hbm.at[pva], yva, semA)
            cpB = pltpu.async_copy(yd_hbm.at[pvb], yvb, semB)
            cpA.wait()
            cpB.wait()
            pltpu.sync_copy(yva, ya_hbm.at[pl.ds(s, CK)])
            pltpu.sync_copy(yvb, yb_hbm.at[pl.ds(s, CK)])

    return body(yd, posA, posB)


def _ffn_body(bexp_ref, xd_ref, w1_ref, b1_ref, w2_ref, b2_ref, yd_ref):
    xb = _unpack_rows(xd_ref[...])
    h = _gelu(jnp.dot(xb, w1_ref[0], preferred_element_type=jnp.float32)
              + b1_ref[0])
    y = (jnp.dot(h, w2_ref[0], preferred_element_type=jnp.float32)
         + b2_ref[0])
    yd_ref[...] = _pack_rows(y)


def _ffn(xd, W1, b1, W2, b2, bexp):
    grid_spec = pltpu.PrefetchScalarGridSpec(
        num_scalar_prefetch=1,
        grid=(NB,),
        in_specs=[
            pl.BlockSpec((B, DW), lambda i, be: (i, 0)),
            pl.BlockSpec((1, D, H), lambda i, be: (be[i], 0, 0)),
            pl.BlockSpec((1, 1, H), lambda i, be: (be[i], 0, 0)),
            pl.BlockSpec((1, H, D), lambda i, be: (be[i], 0, 0)),
            pl.BlockSpec((1, 1, D), lambda i, be: (be[i], 0, 0)),
        ],
        out_specs=pl.BlockSpec((B, DW), lambda i, be: (i, 0)),
    )
    return pl.pallas_call(
        _ffn_body,
        grid_spec=grid_spec,
        out_shape=jax.ShapeDtypeStruct((P, DW), jnp.int32),
    )(bexp, xd, W1, b1.reshape(E, 1, H), W2, b2.reshape(E, 1, D))


def _finish_body(x_ref, ya_ref, yb_ref, wts_ref, ws_ref, bs_ref, g_ref,
                 be_ref, o_ref):
    shared = _gelu(jnp.dot(x_ref[...], ws_ref[...],
                           preferred_element_type=jnp.float32)
                   + bs_ref[...][None, :])
    w = wts_ref[...]
    out = (w[:, 0:1] * _unpack_rows(ya_ref[...])
           + w[:, 1:2] * _unpack_rows(yb_ref[...])
           + 0.5 * shared)
    mu = jnp.mean(out, axis=-1, keepdims=True)
    d = out - mu
    var = jnp.mean(d * d, axis=-1, keepdims=True)
    o_ref[...] = d * jax.lax.rsqrt(var + 1e-5) * g_ref[...][None, :] + be_ref[...][None, :]


def _finish(x, yA, yB, wts, Ws, bs, gamma, beta):
    full = lambda shape: pl.BlockSpec(shape, lambda i: (0,) * len(shape))
    return pl.pallas_call(
        _finish_body,
        grid=(N // TB,),
        in_specs=[
            pl.BlockSpec((TB, D), lambda i: (i, 0)),
            pl.BlockSpec((TB, DW), lambda i: (i, 0)),
            pl.BlockSpec((TB, DW), lambda i: (i, 0)),
            pl.BlockSpec((TB, TOPK), lambda i: (i, 0)),
            full((D, D)),
            full((D,)),
            full((D,)),
            full((D,)),
        ],
        out_specs=pl.BlockSpec((TB, D), lambda i: (i, 0)),
        out_shape=jax.ShapeDtypeStruct((N, D), jnp.float32),
    )(x, yA, yB, wts, Ws, bs, gamma, beta)


@jax.jit
def kernel(x, gate_W, W1, b1, W2, b2, Ws, bs, gamma, beta):
    wts, eid, rnk, cnts8, xh32 = _gate(x, gate_W)
    posA2, posB2, bexp2 = _route(eid, rnk, cnts8)
    posA = posA2.reshape(N)
    posB = posB2.reshape(N)
    bexp = bexp2.reshape(NB)
    xd32 = _dispatch(xh32, posA, posB)
    yd32 = _ffn(xd32, W1, b1, W2, b2, bexp)
    yA32, yB32 = _combine_gather(yd32, posA, posB)
    return _finish(x, yA32, yB32, wts, Ws, bs, gamma, beta)


# speedup vs baseline: 4.6137x; 1.0436x over previous
import functools
import jax
import jax.numpy as jnp
from jax import lax
from jax.experimental import pallas as pl
from jax.experimental.pallas import tpu as pltpu
from jax.experimental.pallas import tpu_sc as plsc

N, D, E, H, TOPK = 8192, 768, 8, 512, 2
TB = 256
B = 256
DW = D // 2
P = N * TOPK + E * B
NB = P // B

NC, NS = 2, 16
NW = NC * NS
TPW = N // NW
CK = 128
NCH = TPW // CK


def _gelu(v):
    return 0.5 * v * (1.0 + jax.lax.erf(v * 0.7071067811865476))


def _pack_rows(xf):
    u = jax.lax.bitcast_convert_type(xf, jnp.uint32)
    r = u + 0x7FFF + ((u >> 16) & 1)
    r = (r >> 16).astype(jnp.int32)
    lo = r[:, :DW]
    hi = r[:, DW:]
    return jnp.bitwise_or(lo, hi << 16)


def _unpack_rows(w):
    lo = jax.lax.bitcast_convert_type(w << 16, jnp.float32)
    hi = jax.lax.bitcast_convert_type(
        jnp.bitwise_and(w, jnp.int32(-65536)), jnp.float32)
    return jnp.concatenate([lo, hi], axis=1)


def _top2(logits):
    T = logits.shape[0]
    iota = jax.lax.broadcasted_iota(jnp.int32, (T, E), 1)
    m1 = jnp.max(logits, axis=-1, keepdims=True)
    i1 = jnp.min(jnp.where(logits == m1, iota, E), axis=-1, keepdims=True)
    l2 = jnp.where(iota == i1, -jnp.inf, logits)
    m2 = jnp.max(l2, axis=-1, keepdims=True)
    i2 = jnp.min(jnp.where(l2 == m2, iota, E), axis=-1, keepdims=True)
    r = jnp.exp(m2 - m1)
    w_a = 1.0 / (1.0 + r)
    return w_a, 1.0 - w_a, i1, i2


def _gate_body(x_ref, gw_ref, wts_ref, eid_ref, rnk_ref, cnts_ref, xh_ref, cntv):
    i = pl.program_id(0)

    @pl.when(i == 0)
    def _init():
        cntv[...] = jnp.zeros((1, E), jnp.float32)

    xh_ref[...] = _pack_rows(x_ref[...])
    logits = jnp.dot(x_ref[...], gw_ref[...], preferred_element_type=jnp.float32)
    w_a, w_b, i1, i2 = _top2(logits)
    wts_ref[...] = jnp.concatenate([w_a, w_b], axis=1)
    eid_ref[...] = jnp.concatenate([i1, i2], axis=1)

    laneio = jax.lax.broadcasted_iota(jnp.int32, (TB, E), 1)
    cnt_tok = (laneio == i1).astype(jnp.float32) + (laneio == i2).astype(jnp.float32)
    lt = (jax.lax.broadcasted_iota(jnp.int32, (TB, TB), 0)
          > jax.lax.broadcasted_iota(jnp.int32, (TB, TB), 1)).astype(jnp.float32)
    cumexc = jnp.dot(lt, cnt_tok, preferred_element_type=jnp.float32)
    base = cntv[...]
    r_all = cumexc + base
    rankA = jnp.sum(jnp.where(laneio == i1, r_all, 0.0), axis=1, keepdims=True)
    rankB = jnp.sum(jnp.where(laneio == i2, r_all, 0.0), axis=1, keepdims=True)
    rnk_ref[...] = jnp.concatenate([rankA, rankB], axis=1).astype(jnp.int32)

    newtot = base + jnp.sum(cnt_tok, axis=0, keepdims=True)
    cntv[...] = newtot
    cnts_ref[...] = jnp.broadcast_to(newtot, (E, E)).astype(jnp.int32)


def _gate(x, gate_W):
    return pl.pallas_call(
        _gate_body,
        grid=(N // TB,),
        in_specs=[
            pl.BlockSpec((TB, D), lambda i: (i, 0)),
            pl.BlockSpec((D, E), lambda i: (0, 0)),
        ],
        out_specs=[
            pl.BlockSpec((TB, TOPK), lambda i: (i, 0)),
            pl.BlockSpec((TB, TOPK), lambda i: (i, 0)),
            pl.BlockSpec((TB, TOPK), lambda i: (i, 0)),
            pl.BlockSpec((E, E), lambda i: (0, 0)),
            pl.BlockSpec((TB, DW), lambda i: (i, 0)),
        ],
        out_shape=[
            jax.ShapeDtypeStruct((N, TOPK), jnp.float32),
            jax.ShapeDtypeStruct((N, TOPK), jnp.int32),
            jax.ShapeDtypeStruct((N, TOPK), jnp.int32),
            jax.ShapeDtypeStruct((E, E), jnp.int32),
            jax.ShapeDtypeStruct((N, DW), jnp.int32),
        ],
        scratch_shapes=[pltpu.VMEM((1, E), jnp.float32)],
    )(x, gate_W)


def _route_body(eid_ref, rnk_ref, cnts_ref, pa_ref, pb_ref, bexp_ref):
    c = cnts_ref[0:1, :]
    padded = jnp.bitwise_and(c + (B - 1), ~(B - 1))
    ut = (jax.lax.broadcasted_iota(jnp.int32, (E, E), 0)
          < jax.lax.broadcasted_iota(jnp.int32, (E, E), 1)).astype(jnp.float32)
    off = jnp.dot(padded.astype(jnp.float32), ut,
                  preferred_element_type=jnp.float32).astype(jnp.int32)
    eid = eid_ref[...]
    laneio = jax.lax.broadcasted_iota(jnp.int32, (TB, E), 1)
    offb = jnp.broadcast_to(off, (TB, E))
    offA = jnp.sum(jnp.where(laneio == eid[:, 0:1], offb, 0),
                   axis=1, keepdims=True)
    offB = jnp.sum(jnp.where(laneio == eid[:, 1:2], offb, 0),
                   axis=1, keepdims=True)
    rnk = rnk_ref[...]
    pa_ref[...] = offA + rnk[:, 0:1]
    pb_ref[...] = offB + rnk[:, 1:2]
    jb = jax.lax.broadcasted_iota(jnp.int32, (NB, E), 0) * B
    used = (jb >= jnp.broadcast_to(off, (NB, E))).astype(jnp.int32)
    bexp_ref[...] = jnp.sum(used, axis=1, keepdims=True) - 1


def _route(eid, rnk, cnts8):
    return pl.pallas_call(
        _route_body,
        grid=(N // TB,),
        in_specs=[
            pl.BlockSpec((TB, TOPK), lambda i: (i, 0)),
            pl.BlockSpec((TB, TOPK), lambda i: (i, 0)),
            pl.BlockSpec((E, E), lambda i: (0, 0)),
        ],
        out_specs=[
            pl.BlockSpec((TB, 1), lambda i: (i, 0)),
            pl.BlockSpec((TB, 1), lambda i: (i, 0)),
            pl.BlockSpec((NB, 1), lambda i: (0, 0)),
        ],
        out_shape=[
            jax.ShapeDtypeStruct((N, 1), jnp.int32),
            jax.ShapeDtypeStruct((N, 1), jnp.int32),
            jax.ShapeDtypeStruct((NB, 1), jnp.int32),
        ],
    )(eid, rnk, cnts8)


def _sc_mesh():
    return plsc.VectorSubcoreMesh(core_axis_name="c", subcore_axis_name="s")


def _dispatch(x, posA, posB):
    @functools.partial(
        pl.kernel,
        mesh=_sc_mesh(),
        out_type=jax.ShapeDtypeStruct((P, DW), jnp.int32),
        scratch_types=[
            pltpu.VMEM((TPW, DW), jnp.int32),
            pltpu.VMEM((NCH, CK), jnp.int32),
            pltpu.VMEM((NCH, CK), jnp.int32),
            pltpu.SemaphoreType.DMA,
            pltpu.SemaphoreType.DMA,
        ],
    )
    def body(x_hbm, pa_hbm, pb_hbm, xd_hbm, xv, pva, pvb, semL, semS):
        wid = lax.axis_index("s") * NC + lax.axis_index("c")
        base = wid * TPW
        loads = [
            pltpu.async_copy(x_hbm.at[pl.ds(base, TPW)], xv, semL),
            pltpu.async_copy(pa_hbm.at[wid], pva, semL),
            pltpu.async_copy(pb_hbm.at[wid], pvb, semL),
        ]
        for cp in loads:
            cp.wait()
        stores = []
        for i in range(NCH):
            src = xv.at[pl.ds(i * CK, CK)]
            stores.append(pltpu.async_copy(src, xd_hbm.at[pva.at[i]], semS))
            stores.append(pltpu.async_copy(src, xd_hbm.at[pvb.at[i]], semS))
        for cp in stores:
            cp.wait()

    return body(x, posA, posB)


def _combine_gather(yd, posA, posB):
    @functools.partial(
        pl.kernel,
        mesh=_sc_mesh(),
        out_type=[
            jax.ShapeDtypeStruct((N, DW), jnp.int32),
            jax.ShapeDtypeStruct((N, DW), jnp.int32),
        ],
        scratch_types=[
            pltpu.VMEM((TPW, DW), jnp.int32),
            pltpu.VMEM((NCH, CK), jnp.int32),
            pltpu.VMEM((NCH, CK), jnp.int32),
            pltpu.SemaphoreType.DMA,
            pltpu.SemaphoreType.DMA,
        ],
    )
    def body(yd_hbm, pa_hbm, pb_hbm, ya_hbm, yb_hbm, yv, pva, pvb,
             semL, semS):
        wid = lax.axis_index("s") * NC + lax.axis_index("c")
        base = wid * TPW
        l1 = pltpu.async_copy(pa_hbm.at[wid], pva, semL)
        l2 = pltpu.async_copy(pb_hbm.at[wid], pvb, semL)
        l1.wait()
        l2.wait()
        for pv, out_hbm in ((pva, ya_hbm), (pvb, yb_hbm)):
            gathers = [
                pltpu.async_copy(yd_hbm.at[pv.at[i]],
                                 yv.at[pl.ds(i * CK, CK)], semS)
                for i in range(NCH)
            ]
            for cp in gathers:
                cp.wait()
            pltpu.sync_copy(yv, out_hbm.at[pl.ds(base, TPW)])

    return body(yd, posA, posB)


def _ffn_body(bexp_ref, xd_ref, w1_ref, b1_ref, w2_ref, b2_ref, yd_ref):
    xb = _unpack_rows(xd_ref[...])
    h = _gelu(jnp.dot(xb, w1_ref[0], preferred_element_type=jnp.float32)
              + b1_ref[0])
    y = (jnp.dot(h, w2_ref[0], preferred_element_type=jnp.float32)
         + b2_ref[0])
    yd_ref[...] = _pack_rows(y)


def _ffn(xd, W1, b1, W2, b2, bexp):
    grid_spec = pltpu.PrefetchScalarGridSpec(
        num_scalar_prefetch=1,
        grid=(NB,),
        in_specs=[
            pl.BlockSpec((B, DW), lambda i, be: (i, 0)),
            pl.BlockSpec((1, D, H), lambda i, be: (be[i], 0, 0)),
            pl.BlockSpec((1, 1, H), lambda i, be: (be[i], 0, 0)),
            pl.BlockSpec((1, H, D), lambda i, be: (be[i], 0, 0)),
            pl.BlockSpec((1, 1, D), lambda i, be: (be[i], 0, 0)),
        ],
        out_specs=pl.BlockSpec((B, DW), lambda i, be: (i, 0)),
    )
    return pl.pallas_call(
        _ffn_body,
        grid_spec=grid_spec,
        out_shape=jax.ShapeDtypeStruct((P, DW), jnp.int32),
    )(bexp, xd, W1, b1.reshape(E, 1, H), W2, b2.reshape(E, 1, D))


def _finish_body(x_ref, ya_ref, yb_ref, wts_ref, ws_ref, bs_ref, g_ref,
                 be_ref, o_ref):
    shared = _gelu(jnp.dot(x_ref[...], ws_ref[...],
                           preferred_element_type=jnp.float32)
                   + bs_ref[...][None, :])
    w = wts_ref[...]
    out = (w[:, 0:1] * _unpack_rows(ya_ref[...])
           + w[:, 1:2] * _unpack_rows(yb_ref[...])
           + 0.5 * shared)
    mu = jnp.mean(out, axis=-1, keepdims=True)
    d = out - mu
    var = jnp.mean(d * d, axis=-1, keepdims=True)
    o_ref[...] = d * jax.lax.rsqrt(var + 1e-5) * g_ref[...][None, :] + be_ref[...][None, :]


def _finish(x, yA, yB, wts, Ws, bs, gamma, beta):
    full = lambda shape: pl.BlockSpec(shape, lambda i: (0,) * len(shape))
    return pl.pallas_call(
        _finish_body,
        grid=(N // TB,),
        in_specs=[
            pl.BlockSpec((TB, D), lambda i: (i, 0)),
            pl.BlockSpec((TB, DW), lambda i: (i, 0)),
            pl.BlockSpec((TB, DW), lambda i: (i, 0)),
            pl.BlockSpec((TB, TOPK), lambda i: (i, 0)),
            full((D, D)),
            full((D,)),
            full((D,)),
            full((D,)),
        ],
        out_specs=pl.BlockSpec((TB, D), lambda i: (i, 0)),
        out_shape=jax.ShapeDtypeStruct((N, D), jnp.float32),
    )(x, yA, yB, wts, Ws, bs, gamma, beta)


@jax.jit
def kernel(x, gate_W, W1, b1, W2, b2, Ws, bs, gamma, beta):
    wts, eid, rnk, cnts8, xh32 = _gate(x, gate_W)
    posA2, posB2, bexp2 = _route(eid, rnk, cnts8)
    posA = posA2.reshape(NW, NCH, CK)
    posB = posB2.reshape(NW, NCH, CK)
    bexp = bexp2.reshape(NB)
    xd32 = _dispatch(xh32, posA, posB)
    yd32 = _ffn(xd32, W1, b1, W2, b2, bexp)
    yA32, yB32 = _combine_gather(yd32, posA, posB)
    return _finish(x, yA32, yB32, wts, Ws, bs, gamma, beta)


# speedup vs baseline: 4.6271x; 1.0029x over previous
import functools
import jax
import jax.numpy as jnp
from jax import lax
from jax.experimental import pallas as pl
from jax.experimental.pallas import tpu as pltpu
from jax.experimental.pallas import tpu_sc as plsc

N, D, E, H, TOPK = 8192, 768, 8, 512, 2
TB = 256
B = 256
DW = D // 2
P = N * TOPK + E * B
NB = P // B

NC, NS = 2, 16
NW = NC * NS
TPW = N // NW
CK = 128
NCH = TPW // CK


def _gelu(v):
    return 0.5 * v * (1.0 + jax.lax.erf(v * 0.7071067811865476))


def _pack_rows(xf):
    u = jax.lax.bitcast_convert_type(xf, jnp.uint32)
    r = u + 0x7FFF + ((u >> 16) & 1)
    r = (r >> 16).astype(jnp.int32)
    lo = r[:, :DW]
    hi = r[:, DW:]
    return jnp.bitwise_or(lo, hi << 16)


def _unpack_rows(w):
    lo = jax.lax.bitcast_convert_type(w << 16, jnp.float32)
    hi = jax.lax.bitcast_convert_type(
        jnp.bitwise_and(w, jnp.int32(-65536)), jnp.float32)
    return jnp.concatenate([lo, hi], axis=1)


def _top2(logits):
    T = logits.shape[0]
    iota = jax.lax.broadcasted_iota(jnp.int32, (T, E), 1)
    m1 = jnp.max(logits, axis=-1, keepdims=True)
    i1 = jnp.min(jnp.where(logits == m1, iota, E), axis=-1, keepdims=True)
    l2 = jnp.where(iota == i1, -jnp.inf, logits)
    m2 = jnp.max(l2, axis=-1, keepdims=True)
    i2 = jnp.min(jnp.where(l2 == m2, iota, E), axis=-1, keepdims=True)
    r = jnp.exp(m2 - m1)
    w_a = 1.0 / (1.0 + r)
    return w_a, 1.0 - w_a, i1, i2


def _gate_body(x_ref, gw_ref, wts_ref, eid_ref, rnk_ref, cnts_ref, xh_ref, cntv):
    i = pl.program_id(0)

    @pl.when(i == 0)
    def _init():
        cntv[...] = jnp.zeros((1, E), jnp.float32)

    xh_ref[...] = _pack_rows(x_ref[...])
    logits = jnp.dot(x_ref[...], gw_ref[...], preferred_element_type=jnp.float32)
    w_a, w_b, i1, i2 = _top2(logits)
    wts_ref[...] = jnp.concatenate([w_a, w_b], axis=1)
    eid_ref[...] = jnp.concatenate([i1, i2], axis=1)

    laneio = jax.lax.broadcasted_iota(jnp.int32, (TB, E), 1)
    cnt_tok = (laneio == i1).astype(jnp.float32) + (laneio == i2).astype(jnp.float32)
    lt = (jax.lax.broadcasted_iota(jnp.int32, (TB, TB), 0)
          > jax.lax.broadcasted_iota(jnp.int32, (TB, TB), 1)).astype(jnp.float32)
    cumexc = jnp.dot(lt, cnt_tok, preferred_element_type=jnp.float32)
    base = cntv[...]
    r_all = cumexc + base
    rankA = jnp.sum(jnp.where(laneio == i1, r_all, 0.0), axis=1, keepdims=True)
    rankB = jnp.sum(jnp.where(laneio == i2, r_all, 0.0), axis=1, keepdims=True)
    rnk_ref[...] = jnp.concatenate([rankA, rankB], axis=1).astype(jnp.int32)

    newtot = base + jnp.sum(cnt_tok, axis=0, keepdims=True)
    cntv[...] = newtot
    cnts_ref[...] = jnp.broadcast_to(newtot, (E, E)).astype(jnp.int32)


def _gate(x, gate_W):
    return pl.pallas_call(
        _gate_body,
        grid=(N // TB,),
        in_specs=[
            pl.BlockSpec((TB, D), lambda i: (i, 0)),
            pl.BlockSpec((D, E), lambda i: (0, 0)),
        ],
        out_specs=[
            pl.BlockSpec((TB, TOPK), lambda i: (i, 0)),
            pl.BlockSpec((TB, TOPK), lambda i: (i, 0)),
            pl.BlockSpec((TB, TOPK), lambda i: (i, 0)),
            pl.BlockSpec((E, E), lambda i: (0, 0)),
            pl.BlockSpec((TB, DW), lambda i: (i, 0)),
        ],
        out_shape=[
            jax.ShapeDtypeStruct((N, TOPK), jnp.float32),
            jax.ShapeDtypeStruct((N, TOPK), jnp.int32),
            jax.ShapeDtypeStruct((N, TOPK), jnp.int32),
            jax.ShapeDtypeStruct((E, E), jnp.int32),
            jax.ShapeDtypeStruct((N, DW), jnp.int32),
        ],
        scratch_shapes=[pltpu.VMEM((1, E), jnp.float32)],
    )(x, gate_W)


def _route_body(eid_ref, rnk_ref, cnts_ref, pa_ref, pb_ref, bexp_ref):
    c = cnts_ref[0:1, :]
    padded = jnp.bitwise_and(c + (B - 1), ~(B - 1))
    ut = (jax.lax.broadcasted_iota(jnp.int32, (E, E), 0)
          < jax.lax.broadcasted_iota(jnp.int32, (E, E), 1)).astype(jnp.float32)
    off = jnp.dot(padded.astype(jnp.float32), ut,
                  preferred_element_type=jnp.float32).astype(jnp.int32)
    eid = eid_ref[...]
    laneio = jax.lax.broadcasted_iota(jnp.int32, (TB, E), 1)
    offb = jnp.broadcast_to(off, (TB, E))
    offA = jnp.sum(jnp.where(laneio == eid[:, 0:1], offb, 0),
                   axis=1, keepdims=True)
    offB = jnp.sum(jnp.where(laneio == eid[:, 1:2], offb, 0),
                   axis=1, keepdims=True)
    rnk = rnk_ref[...]
    pa_ref[...] = offA + rnk[:, 0:1]
    pb_ref[...] = offB + rnk[:, 1:2]
    jb = jax.lax.broadcasted_iota(jnp.int32, (NB, E), 0) * B
    used = (jb >= jnp.broadcast_to(off, (NB, E))).astype(jnp.int32)
    bexp_ref[...] = jnp.sum(used, axis=1, keepdims=True) - 1


def _route(eid, rnk, cnts8):
    return pl.pallas_call(
        _route_body,
        grid=(N // TB,),
        in_specs=[
            pl.BlockSpec((TB, TOPK), lambda i: (i, 0)),
            pl.BlockSpec((TB, TOPK), lambda i: (i, 0)),
            pl.BlockSpec((E, E), lambda i: (0, 0)),
        ],
        out_specs=[
            pl.BlockSpec((TB, 1), lambda i: (i, 0)),
            pl.BlockSpec((TB, 1), lambda i: (i, 0)),
            pl.BlockSpec((NB, 1), lambda i: (0, 0)),
        ],
        out_shape=[
            jax.ShapeDtypeStruct((N, 1), jnp.int32),
            jax.ShapeDtypeStruct((N, 1), jnp.int32),
            jax.ShapeDtypeStruct((NB, 1), jnp.int32),
        ],
    )(eid, rnk, cnts8)


def _sc_mesh():
    return plsc.VectorSubcoreMesh(core_axis_name="c", subcore_axis_name="s")


def _dispatch(x, posA, posB):
    @functools.partial(
        pl.kernel,
        mesh=_sc_mesh(),
        out_type=jax.ShapeDtypeStruct((P, DW), jnp.int32),
        scratch_types=[
            pltpu.VMEM((TPW, DW), jnp.int32),
            pltpu.VMEM((NCH, CK), jnp.int32),
            pltpu.VMEM((NCH, CK), jnp.int32),
            pltpu.SemaphoreType.DMA,
            pltpu.SemaphoreType.DMA,
        ],
    )
    def body(x_hbm, pa_hbm, pb_hbm, xd_hbm, xv, pva, pvb, semL, semS):
        wid = lax.axis_index("s") * NC + lax.axis_index("c")
        base = wid * TPW
        loads = [
            pltpu.async_copy(x_hbm.at[pl.ds(base, TPW)], xv, semL),
            pltpu.async_copy(pa_hbm.at[wid], pva, semL),
            pltpu.async_copy(pb_hbm.at[wid], pvb, semL),
        ]
        for cp in loads:
            cp.wait()
        stores = []
        for i in range(NCH):
            src = xv.at[pl.ds(i * CK, CK)]
            stores.append(pltpu.async_copy(src, xd_hbm.at[pva.at[i]], semS))
            stores.append(pltpu.async_copy(src, xd_hbm.at[pvb.at[i]], semS))
        for cp in stores:
            cp.wait()

    return body(x, posA, posB)


def _combine_gather(yd, posA, posB):
    @functools.partial(
        pl.kernel,
        mesh=_sc_mesh(),
        out_type=[
            jax.ShapeDtypeStruct((N, DW), jnp.int32),
            jax.ShapeDtypeStruct((N, DW), jnp.int32),
        ],
        scratch_types=[
            pltpu.VMEM((TPW, DW), jnp.int32),
            pltpu.VMEM((NCH, CK), jnp.int32),
            pltpu.VMEM((NCH, CK), jnp.int32),
            pltpu.SemaphoreType.DMA,
            pltpu.SemaphoreType.DMA,
        ],
    )
    def body(yd_hbm, pa_hbm, pb_hbm, ya_hbm, yb_hbm, yv, pva, pvb,
             semL, semS):
        wid = lax.axis_index("s") * NC + lax.axis_index("c")
        base = wid * TPW
        l1 = pltpu.async_copy(pa_hbm.at[wid], pva, semL)
        l2 = pltpu.async_copy(pb_hbm.at[wid], pvb, semL)
        l1.wait()
        l2.wait()
        for pv, out_hbm in ((pva, ya_hbm), (pvb, yb_hbm)):
            gathers = [
                pltpu.async_copy(yd_hbm.at[pv.at[i]],
                                 yv.at[pl.ds(i * CK, CK)], semS)
                for i in range(NCH)
            ]
            for cp in gathers:
                cp.wait()
            pltpu.sync_copy(yv, out_hbm.at[pl.ds(base, TPW)])

    return body(yd, posA, posB)


def _ffn_body(bexp_ref, xd_ref, w1_ref, b1_ref, w2_ref, b2_ref, yd_ref):
    xb = _unpack_rows(xd_ref[...]).astype(jnp.bfloat16)
    h = _gelu(jnp.dot(xb, w1_ref[0].astype(jnp.bfloat16),
                      preferred_element_type=jnp.float32)
              + b1_ref[0])
    y = (jnp.dot(h.astype(jnp.bfloat16), w2_ref[0].astype(jnp.bfloat16),
                 preferred_element_type=jnp.float32)
         + b2_ref[0])
    yd_ref[...] = _pack_rows(y)


def _ffn(xd, W1, b1, W2, b2, bexp):
    grid_spec = pltpu.PrefetchScalarGridSpec(
        num_scalar_prefetch=1,
        grid=(NB,),
        in_specs=[
            pl.BlockSpec((B, DW), lambda i, be: (i, 0)),
            pl.BlockSpec((1, D, H), lambda i, be: (be[i], 0, 0)),
            pl.BlockSpec((1, 1, H), lambda i, be: (be[i], 0, 0)),
            pl.BlockSpec((1, H, D), lambda i, be: (be[i], 0, 0)),
            pl.BlockSpec((1, 1, D), lambda i, be: (be[i], 0, 0)),
        ],
        out_specs=pl.BlockSpec((B, DW), lambda i, be: (i, 0)),
    )
    return pl.pallas_call(
        _ffn_body,
        grid_spec=grid_spec,
        out_shape=jax.ShapeDtypeStruct((P, DW), jnp.int32),
    )(bexp, xd, W1, b1.reshape(E, 1, H), W2, b2.reshape(E, 1, D))


def _finish_body(x_ref, ya_ref, yb_ref, wts_ref, ws_ref, bs_ref, g_ref,
                 be_ref, o_ref):
    shared = _gelu(jnp.dot(x_ref[...].astype(jnp.bfloat16),
                           ws_ref[...].astype(jnp.bfloat16),
                           preferred_element_type=jnp.float32)
                   + bs_ref[...][None, :])
    w = wts_ref[...]
    out = (w[:, 0:1] * _unpack_rows(ya_ref[...])
           + w[:, 1:2] * _unpack_rows(yb_ref[...])
           + 0.5 * shared)
    mu = jnp.mean(out, axis=-1, keepdims=True)
    d = out - mu
    var = jnp.mean(d * d, axis=-1, keepdims=True)
    o_ref[...] = d * jax.lax.rsqrt(var + 1e-5) * g_ref[...][None, :] + be_ref[...][None, :]


def _finish(x, yA, yB, wts, Ws, bs, gamma, beta):
    full = lambda shape: pl.BlockSpec(shape, lambda i: (0,) * len(shape))
    return pl.pallas_call(
        _finish_body,
        grid=(N // TB,),
        in_specs=[
            pl.BlockSpec((TB, D), lambda i: (i, 0)),
            pl.BlockSpec((TB, DW), lambda i: (i, 0)),
            pl.BlockSpec((TB, DW), lambda i: (i, 0)),
            pl.BlockSpec((TB, TOPK), lambda i: (i, 0)),
            full((D, D)),
            full((D,)),
            full((D,)),
            full((D,)),
        ],
        out_specs=pl.BlockSpec((TB, D), lambda i: (i, 0)),
        out_shape=jax.ShapeDtypeStruct((N, D), jnp.float32),
    )(x, yA, yB, wts, Ws, bs, gamma, beta)


@jax.jit
def kernel(x, gate_W, W1, b1, W2, b2, Ws, bs, gamma, beta):
    wts, eid, rnk, cnts8, xh32 = _gate(x, gate_W)
    posA2, posB2, bexp2 = _route(eid, rnk, cnts8)
    posA = posA2.reshape(NW, NCH, CK)
    posB = posB2.reshape(NW, NCH, CK)
    bexp = bexp2.reshape(NB)
    xd32 = _dispatch(xh32, posA, posB)
    yd32 = _ffn(xd32, W1, b1, W2, b2, bexp)
    yA32, yB32 = _combine_gather(yd32, posA, posB)
    return _finish(x, yA32, yB32, wts, Ws, bs, gamma, beta)
